# Initial kernel scaffold; baseline (speedup 1.0000x reference)
import jax
import jax.numpy as jnp
from jax.experimental import pallas as pl


def kernel(c_feat, edge_idx, edge_attr, v_feat, batch_idx, params):
    raise NotImplementedError("write your pallas kernel here")


# speedup vs baseline: 2.5898x; 2.5898x over previous
import functools

import jax
import jax.numpy as jnp
import numpy as np
from jax import lax
from jax.experimental import pallas as pl
from jax.experimental.pallas import tpu as pltpu
from jax.experimental.pallas import tpu_sc as plsc

EMB = 64
NC = 10000
NV = 10000
NEDGE = 320000
NB = 16
NE = 16
TOPK = 4
TEMP = 0.6
NT = 64
TD = 64
HID = EMB * 4

_NCORES = 2
_NSUB = 16
_NW = _NCORES * _NSUB
_EPW = NEDGE // _NW
_CHUNK = 80
_NCHUNK = _EPW // _CHUNK
_RPS = NC // _NSUB


def _edge_pass_body(table_h, gidx_h, sidx_h, a_h, r_h, z_h, out_h,
                    idx_g, idx_s, a_v, rows, r_v, acc, sem):
    cid = lax.axis_index("c")
    sid = lax.axis_index("s")
    wid = sid * _NCORES + cid

    @pl.when(sid == 0)
    def _():
        pltpu.sync_copy(z_h, acc)

    pltpu.sync_copy(r_h, r_v)
    plsc.subcore_barrier()

    rv = [r_v[pl.ds(16 * d, 16)] for d in range(4)]
    base0 = wid * _EPW

    def chunk_body(c, carry):
        base = base0 + c * _CHUNK
        pltpu.sync_copy(gidx_h.at[pl.ds(base, _CHUNK)], idx_g)
        pltpu.sync_copy(sidx_h.at[pl.ds(base, _CHUNK)], idx_s)
        pltpu.sync_copy(a_h.at[pl.ds(base, _CHUNK)], a_v)
        pltpu.async_copy(table_h.at[idx_g], rows, sem).wait()

        def edge_body(i, carry2):
            av = plsc.load_gather(a_v, [jnp.full((16,), i, dtype=jnp.int32)])
            for d in range(4):
                seg = rows[i, pl.ds(16 * d, 16)]
                rows[i, pl.ds(16 * d, 16)] = jnp.maximum(seg + av * rv[d], 0.0)
            return carry2

        lax.fori_loop(0, _CHUNK, edge_body, 0)
        pltpu.sync_copy(rows, acc.at[idx_s], add=True)
        return carry

    lax.fori_loop(0, _NCHUNK, chunk_body, 0)
    plsc.subcore_barrier()

    @pl.when(sid == 0)
    def _():
        pltpu.sync_copy(acc, out_h.at[cid])


@jax.jit
def _edge_pass(table, gidx, sidx, a, r):
    z = jnp.zeros((NC, EMB), jnp.float32)
    f = pl.kernel(
        _edge_pass_body,
        mesh=plsc.VectorSubcoreMesh(core_axis_name="c", subcore_axis_name="s"),
        out_type=jax.ShapeDtypeStruct((_NCORES, NC, EMB), jnp.float32),
        scratch_types=[
            pltpu.VMEM((_CHUNK,), jnp.int32),
            pltpu.VMEM((_CHUNK,), jnp.int32),
            pltpu.VMEM((_CHUNK,), jnp.float32),
            pltpu.VMEM((_CHUNK, EMB), jnp.float32),
            pltpu.VMEM((EMB,), jnp.float32),
            pltpu.VMEM_SHARED((NC, EMB), jnp.float32),
            pltpu.SemaphoreType.DMA,
        ],
        compiler_params=pltpu.CompilerParams(needs_layout_passes=False,
                                             use_tc_tiling_on_sc=False),
    )
    return f(table, gidx, sidx, a, r, z)


def _ln(x, g, b):
    mu = jnp.mean(x, axis=-1, keepdims=True)
    var = jnp.mean((x - mu) ** 2, axis=-1, keepdims=True)
    return (x - mu) / jnp.sqrt(var + 1e-5) * g + b


def _expert(x, W1, b1, W2, b2, g, be):
    h = jax.nn.gelu(x @ W1 + b1, approximate=False)
    return _ln(h @ W2 + b2, g, be)


def kernel(c_feat, edge_idx, edge_attr, v_feat, batch_idx, params):
    p = params
    ci = edge_idx[0]
    vi = edge_idx[1]
    a = edge_attr[:, 0]

    c0 = jax.nn.relu(c_feat @ p['Wc'] + p['bc'])
    v0 = jax.nn.relu(v_feat @ p['Wv'] + p['bv'])

    r1 = p['We'][0] @ p['Wm1']
    k1 = p['be'] @ p['Wm1'] + p['bm1']
    v0mk = v0 @ p['Wm1'] + k1

    s1p = _edge_pass(v0mk, vi, ci, a, r1)
    s1 = s1p[0] + s1p[1]
    c1 = jax.nn.relu(c0 + s1 @ p['Wu1'] + p['bu1'])

    r2 = p['We'][0] @ p['Wm2']
    k2 = p['be'] @ p['Wm2'] + p['bm2']
    c1mk = c1 @ p['Wm2'] + k2

    s2p = _edge_pass(c1mk, ci, vi, a, r2)
    s2 = s2p[0] + s2p[1]
    v1 = jax.nn.relu(v0 + s2 @ p['Wu2'] + p['bu2'])

    onehot = (batch_idx[:, None] == jnp.arange(NB)[None, :]).astype(jnp.float32)
    cnt = onehot.sum(0)
    g_emb = (onehot.T @ v1) / jnp.clip(cnt, 1.0)[:, None]
    Q = v1 @ p['Wq'] + p['bq']
    scores = Q @ p['tok_K'].T / np.sqrt(TD).astype(np.float32)
    w_tok = jax.nn.softmax(scores, axis=-1)
    node_struct = w_tok @ p['tok_V']
    struct_emb = (onehot.T @ node_struct) / jnp.clip(cnt, 1.0)[:, None]

    gate_in = jnp.concatenate([g_emb, struct_emb], axis=-1)
    logits = (gate_in @ p['Wg'] + p['bg']) * p['alpha'] / TEMP + p['ebias'][None]

    l = logits
    mask = jnp.zeros_like(l)
    iota = jnp.arange(NE)[None, :].astype(jnp.float32) * jnp.ones((NB, 1))
    for _ in range(TOPK):
        m = jnp.max(l, axis=-1, keepdims=True)
        is_max = (l == m)
        first = jnp.min(jnp.where(is_max, iota, 1e9), axis=-1, keepdims=True)
        fm = (iota == first).astype(l.dtype)
        mask = mask + fm
        l = jnp.where(fm > 0, -1e30, l)
    rw = jax.nn.softmax(logits, axis=-1) * mask
    rw = rw / (rw.sum(axis=-1, keepdims=True) + 1e-12)

    shared = (_expert(v1, p['sW1'][0], p['sb1'][0], p['sW2'][0], p['sb2'][0],
                      p['sg'][0], p['sbe'][0])
              + _expert(v1, p['sW1'][1], p['sb1'][1], p['sW2'][1], p['sb2'][1],
                        p['sg'][1], p['sbe'][1])) * 0.5

    rw_node = onehot @ rw
    fused = jnp.zeros_like(v1)
    for e in range(NE):
        oe = _expert(v1, p['dW1'][e], p['db1'][e], p['dW2'][e], p['db2'][e],
                     p['dg'][e], p['dbe'][e])
        fused = fused + rw_node[:, e][:, None] * oe

    h = v1 + shared + fused
    out = jax.nn.relu(h @ p['Wd1'] + p['bd1']) @ p['Wd2'] + p['bd2']
    return out[:, 0]


# speedup vs baseline: 3.6693x; 1.4168x over previous
import functools

import jax
import jax.numpy as jnp
import numpy as np
from jax import lax
from jax.experimental import pallas as pl
from jax.experimental.pallas import tpu as pltpu
from jax.experimental.pallas import tpu_sc as plsc

EMB = 64
NC = 10000
NV = 10000
NEDGE = 320000
NB = 16
NE = 16
TOPK = 4
TEMP = 0.6
NT = 64
TD = 64
HID = EMB * 4
KS_ = 2

_NCORES = 2
_NSUB = 16
_NW = _NCORES * _NSUB
_EPW = NEDGE // _NW
_CHUNK = 80
_NCHUNK = _EPW // _CHUNK

_TP = 2000
_TF = 200


def _edge_pass_body(table_h, gidx_h, sidx_h, a_h, r_h, z_h, out_h,
                    idx_g, idx_s, a_v, rows, r_v, acc, sem):
    cid = lax.axis_index("c")
    sid = lax.axis_index("s")
    wid = sid * _NCORES + cid

    @pl.when(sid == 0)
    def _():
        pltpu.sync_copy(z_h, acc)

    pltpu.sync_copy(r_h, r_v)
    plsc.subcore_barrier()

    rv = [r_v[pl.ds(16 * d, 16)] for d in range(4)]
    base0 = wid * _EPW

    def chunk_body(c, carry):
        base = base0 + c * _CHUNK
        pltpu.sync_copy(gidx_h.at[pl.ds(base, _CHUNK)], idx_g)
        pltpu.sync_copy(sidx_h.at[pl.ds(base, _CHUNK)], idx_s)
        pltpu.sync_copy(a_h.at[pl.ds(base, _CHUNK)], a_v)
        pltpu.async_copy(table_h.at[idx_g], rows, sem).wait()

        def edge_body(i, carry2):
            av = plsc.load_gather(a_v, [jnp.full((16,), i, dtype=jnp.int32)])
            for d in range(4):
                seg = rows[i, pl.ds(16 * d, 16)]
                rows[i, pl.ds(16 * d, 16)] = jnp.maximum(seg + av * rv[d], 0.0)
            return carry2

        lax.fori_loop(0, _CHUNK, edge_body, 0)
        pltpu.sync_copy(rows, acc.at[idx_s], add=True)
        return carry

    lax.fori_loop(0, _NCHUNK, chunk_body, 0)
    plsc.subcore_barrier()

    @pl.when(sid == 0)
    def _():
        pltpu.sync_copy(acc, out_h.at[cid])


def _edge_pass(table, gidx, sidx, a, r):
    z = jnp.zeros((NC, EMB), jnp.float32)
    f = pl.kernel(
        _edge_pass_body,
        mesh=plsc.VectorSubcoreMesh(core_axis_name="c", subcore_axis_name="s"),
        out_type=jax.ShapeDtypeStruct((_NCORES, NC, EMB), jnp.float32),
        scratch_types=[
            pltpu.VMEM((_CHUNK,), jnp.int32),
            pltpu.VMEM((_CHUNK,), jnp.int32),
            pltpu.VMEM((_CHUNK,), jnp.float32),
            pltpu.VMEM((_CHUNK, EMB), jnp.float32),
            pltpu.VMEM((EMB,), jnp.float32),
            pltpu.VMEM_SHARED((NC, EMB), jnp.float32),
            pltpu.SemaphoreType.DMA,
        ],
        compiler_params=pltpu.CompilerParams(needs_layout_passes=False,
                                             use_tc_tiling_on_sc=False),
    )
    return f(table, gidx, sidx, a, r, z)


def _mm(x, w):
    return jax.lax.dot_general(x, w, (((1,), (0,)), ((), ())),
                               preferred_element_type=jnp.float32)


def _mmT(x, w):
    return jax.lax.dot_general(x, w, (((1,), (1,)), ((), ())),
                               preferred_element_type=jnp.float32)


def _mm0(x, w):
    return jax.lax.dot_general(x, w, (((0,), (0,)), ((), ())),
                               preferred_element_type=jnp.float32)


def _prep_body(cf_ref, vf_ref, Wc_ref, bc_ref, Wv_ref, bv_ref, Wm1_ref,
               k1_ref, c0_ref, v0_ref, v0mk_ref):
    c0 = jnp.maximum(_mm(cf_ref[...], Wc_ref[...]) + bc_ref[...], 0.0)
    v0 = jnp.maximum(_mm(vf_ref[...], Wv_ref[...]) + bv_ref[...], 0.0)
    c0_ref[...] = c0
    v0_ref[...] = v0
    v0mk_ref[...] = _mm(v0, Wm1_ref[...]) + k1_ref[...]


def _mid_body(s1p_ref, c0_ref, Wu1_ref, bu1_ref, Wm2_ref, k2_ref, c1mk_ref):
    s1 = s1p_ref[0] + s1p_ref[1]
    c1 = jnp.maximum(c0_ref[...] + _mm(s1, Wu1_ref[...]) + bu1_ref[...], 0.0)
    c1mk_ref[...] = _mm(c1, Wm2_ref[...]) + k2_ref[...]


def _node_body(s2p_ref, v0_ref, b3_ref, Wu2_ref, bu2_ref, Wq_ref, bq_ref,
               tokK_ref, tokV_ref, v1_ref, acc3_ref):
    s2 = s2p_ref[0] + s2p_ref[1]
    v1 = jnp.maximum(v0_ref[...] + _mm(s2, Wu2_ref[...]) + bu2_ref[...], 0.0)
    v1_ref[...] = v1

    q = _mm(v1, Wq_ref[...]) + bq_ref[...]
    s = _mmT(q, tokK_ref[...]) * (1.0 / np.sqrt(TD).astype(np.float32))
    m = jnp.max(s, axis=-1, keepdims=True)
    e = jnp.exp(s - m)
    w = e / jnp.sum(e, axis=-1, keepdims=True)
    ns = _mm(w, tokV_ref[...])

    b = b3_ref[0, 0, :]
    oh = (b[:, None] == lax.broadcasted_iota(jnp.int32, (_TP, NB), 1)
          ).astype(jnp.float32)

    @pl.when(pl.program_id(0) == 0)
    def _():
        acc3_ref[...] = jnp.zeros_like(acc3_ref)

    acc3_ref[0] += _mm0(oh, v1)
    acc3_ref[1] += _mm0(oh, ns)
    acc3_ref[2] += _mm0(oh, jnp.ones((_TP, EMB), jnp.float32))


def _gate_body(acc3_ref, Wg_ref, bg_ref, rw_ref):
    cnt = jnp.maximum(acc3_ref[2], 1.0)
    g_emb = acc3_ref[0] / cnt
    st_emb = acc3_ref[1] / cnt
    gate_in = jnp.concatenate([g_emb, st_emb], axis=-1)
    logits = _mm(gate_in, Wg_ref[...]) + bg_ref[...]

    l = logits
    mask = jnp.zeros_like(l)
    iota = lax.broadcasted_iota(jnp.int32, (NB, NE), 1)
    for _ in range(TOPK):
        m = jnp.max(l, axis=-1, keepdims=True)
        is_max = l == m
        first = jnp.min(jnp.where(is_max, iota, NE), axis=-1, keepdims=True)
        fm = (iota == first).astype(l.dtype)
        mask = mask + fm
        l = jnp.where(fm > 0.5, -1e30, l)

    mx = jnp.max(logits, axis=-1, keepdims=True)
    ex = jnp.exp(logits - mx)
    sm = ex / jnp.sum(ex, axis=-1, keepdims=True)
    rw = sm * mask
    rw_ref[...] = rw / (jnp.sum(rw, axis=-1, keepdims=True) + 1e-12)


def _gelu(x):
    return x * 0.5 * (1.0 + lax.erf(x * np.float32(1.0 / np.sqrt(2.0))))


def _ln_rows(o, g, b):
    mu = jnp.mean(o, axis=-1, keepdims=True)
    var = jnp.mean((o - mu) ** 2, axis=-1, keepdims=True)
    return (o - mu) * jax.lax.rsqrt(var + 1e-5) * g + b


def _moe_body(v1_ref, b3_ref, rw_ref, sW1_ref, sb1_ref, sW2_ref, sb2_ref,
              sg_ref, sbe_ref, dW1_ref, db1_ref, dW2_ref, db2_ref, dg_ref,
              dbe_ref, Wd1_ref, bd1_ref, Wd2_ref, bd2_ref, out_ref, acc_ref):
    x = v1_ref[...]
    b = b3_ref[0, 0, :]
    oh = (b[:, None] == lax.broadcasted_iota(jnp.int32, (_TF, NB), 1)
          ).astype(jnp.float32)
    rw_node = _mm(oh, rw_ref[...])

    sh = jnp.zeros((_TF, EMB), jnp.float32)
    for s in range(2):
        h = _gelu(_mm(x, sW1_ref[s]) + sb1_ref[pl.ds(s, 1), :])
        o = _mm(h, sW2_ref[s]) + sb2_ref[pl.ds(s, 1), :]
        sh = sh + _ln_rows(o, sg_ref[pl.ds(s, 1), :], sbe_ref[pl.ds(s, 1), :])
    acc_ref[...] = x + sh * 0.5

    for e_i in range(NE):
        w_e = rw_node[:, e_i:e_i + 1]

        @pl.when(jnp.max(w_e) > 0.0)
        def _():
            h = _gelu(_mm(x, dW1_ref[e_i]) + db1_ref[pl.ds(e_i, 1), :])
            o = _mm(h, dW2_ref[e_i]) + db2_ref[pl.ds(e_i, 1), :]
            o = _ln_rows(o, dg_ref[pl.ds(e_i, 1), :], dbe_ref[pl.ds(e_i, 1), :])
            acc_ref[...] += w_e * o

    hfin = acc_ref[...]
    hd = jnp.maximum(_mm(hfin, Wd1_ref[...]) + bd1_ref[...], 0.0)
    out_ref[...] = _mm(hd, Wd2_ref[...]) + bd2_ref[...]


def _full(shape):
    return pl.BlockSpec(shape, lambda i: tuple(0 for _ in shape))


def kernel(c_feat, edge_idx, edge_attr, v_feat, batch_idx, params):
    p = params
    ci = edge_idx[0]
    vi = edge_idx[1]
    a = edge_attr[:, 0]

    f32 = jnp.float32
    cf8 = jnp.pad(c_feat, ((0, 0), (0, 4)))
    vf8 = jnp.pad(v_feat, ((0, 0), (0, 2)))
    Wc8 = jnp.pad(p['Wc'], ((0, 4), (0, 0)))
    Wv8 = jnp.pad(p['Wv'], ((0, 2), (0, 0)))

    r1 = p['We'][0] @ p['Wm1']
    k1 = (p['be'] @ p['Wm1'] + p['bm1']).reshape(1, EMB)
    r2 = p['We'][0] @ p['Wm2']
    k2 = (p['be'] @ p['Wm2'] + p['bm2']).reshape(1, EMB)

    nblk = NC // _TP
    grid = (nblk,)

    c0, v0, v0mk = pl.pallas_call(
        _prep_body,
        grid=grid,
        in_specs=[
            pl.BlockSpec((_TP, 8), lambda i: (i, 0)),
            pl.BlockSpec((_TP, 8), lambda i: (i, 0)),
            _full((8, EMB)), _full((1, EMB)),
            _full((8, EMB)), _full((1, EMB)),
            _full((EMB, EMB)), _full((1, EMB)),
        ],
        out_specs=[
            pl.BlockSpec((_TP, EMB), lambda i: (i, 0)),
            pl.BlockSpec((_TP, EMB), lambda i: (i, 0)),
            pl.BlockSpec((_TP, EMB), lambda i: (i, 0)),
        ],
        out_shape=[jax.ShapeDtypeStruct((NC, EMB), f32)] * 3,
    )(cf8, vf8, Wc8, p['bc'].reshape(1, EMB), Wv8, p['bv'].reshape(1, EMB),
      p['Wm1'], k1)

    s1p = _edge_pass(v0mk, vi, ci, a, r1)

    c1mk = pl.pallas_call(
        _mid_body,
        grid=grid,
        in_specs=[
            pl.BlockSpec((_NCORES, _TP, EMB), lambda i: (0, i, 0)),
            pl.BlockSpec((_TP, EMB), lambda i: (i, 0)),
            _full((EMB, EMB)), _full((1, EMB)),
            _full((EMB, EMB)), _full((1, EMB)),
        ],
        out_specs=pl.BlockSpec((_TP, EMB), lambda i: (i, 0)),
        out_shape=jax.ShapeDtypeStruct((NC, EMB), f32),
    )(s1p, c0, p['Wu1'], p['bu1'].reshape(1, EMB), p['Wm2'], k2)

    s2p = _edge_pass(c1mk, ci, vi, a, r2)

    b3p = batch_idx.reshape(nblk, 1, _TP)
    v1, acc3 = pl.pallas_call(
        _node_body,
        grid=grid,
        in_specs=[
            pl.BlockSpec((_NCORES, _TP, EMB), lambda i: (0, i, 0)),
            pl.BlockSpec((_TP, EMB), lambda i: (i, 0)),
            pl.BlockSpec((1, 1, _TP), lambda i: (i, 0, 0)),
            _full((EMB, EMB)), _full((1, EMB)),
            _full((EMB, TD)), _full((1, TD)),
            _full((NT, TD)), _full((NT, TD)),
        ],
        out_specs=[
            pl.BlockSpec((_TP, EMB), lambda i: (i, 0)),
            pl.BlockSpec((3, NB, EMB), lambda i: (0, 0, 0)),
        ],
        out_shape=[
            jax.ShapeDtypeStruct((NV, EMB), f32),
            jax.ShapeDtypeStruct((3, NB, EMB), f32),
        ],
    )(s2p, v0, b3p, p['Wu2'], p['bu2'].reshape(1, EMB), p['Wq'],
      p['bq'].reshape(1, TD), p['tok_K'], p['tok_V'])

    scale = p['alpha'] / TEMP
    WgE = p['Wg'] * scale
    bgE = (p['bg'] * scale + p['ebias']).reshape(1, NE)

    rw = pl.pallas_call(
        _gate_body,
        grid=(1,),
        in_specs=[_full((3, NB, EMB)), _full((EMB + TD, NE)), _full((1, NE))],
        out_specs=_full((NB, NE)),
        out_shape=jax.ShapeDtypeStruct((NB, NE), f32),
    )(acc3, WgE, bgE)

    nblk_f = NV // _TF
    b3f = batch_idx.reshape(nblk_f, 1, _TF)
    Wd2p = jnp.pad(p['Wd2'], ((0, 0), (0, 127)))
    bd2p = jnp.pad(p['bd2'].reshape(1, 1), ((0, 0), (0, 127)))

    outp = pl.pallas_call(
        _moe_body,
        grid=(nblk_f,),
        in_specs=[
            pl.BlockSpec((_TF, EMB), lambda i: (i, 0)),
            pl.BlockSpec((1, 1, _TF), lambda i: (i, 0, 0)),
            _full((NB, NE)),
            _full((KS_, EMB, HID)), _full((KS_, HID)),
            _full((KS_, HID, EMB)), _full((KS_, EMB)),
            _full((KS_, EMB)), _full((KS_, EMB)),
            _full((NE, EMB, HID)), _full((NE, HID)),
            _full((NE, HID, EMB)), _full((NE, EMB)),
            _full((NE, EMB)), _full((NE, EMB)),
            _full((EMB, EMB)), _full((1, EMB)),
            _full((EMB, 128)), _full((1, 128)),
        ],
        out_specs=pl.BlockSpec((_TF, 128), lambda i: (i, 0)),
        out_shape=jax.ShapeDtypeStruct((NV, 128), f32),
        scratch_shapes=[pltpu.VMEM((_TF, EMB), f32)],
    )(v1, b3f, rw, p['sW1'], p['sb1'], p['sW2'], p['sb2'], p['sg'], p['sbe'],
      p['dW1'], p['db1'], p['dW2'], p['db2'], p['dg'], p['dbe'],
      p['Wd1'], p['bd1'].reshape(1, EMB), Wd2p, bd2p)

    return outp[:, 0]


# speedup vs baseline: 4.9572x; 1.3510x over previous
import functools

import jax
import jax.numpy as jnp
import numpy as np
from jax import lax
from jax.experimental import pallas as pl
from jax.experimental.pallas import tpu as pltpu
from jax.experimental.pallas import tpu_sc as plsc

EMB = 64
NC = 10000
NV = 10000
NEDGE = 320000
NB = 16
NE = 16
TOPK = 4
TEMP = 0.6
NT = 64
TD = 64
HID = EMB * 4
KS_ = 2

_NCORES = 2
_NSUB = 16
_NW = _NCORES * _NSUB
_EPW = NEDGE // _NW
_CHUNK = 80
_NCHUNK = _EPW // _CHUNK

_TP = 2000
_TF = 200


_NBUF = 3
_MAIN = (_NCHUNK // _NBUF) * _NBUF - _NBUF


def _edge_pass_body(table_h, gidx_h, sidx_h, a_h, r_h, z_h, out_h,
                    idx_g, idx_s, a_v, rows, r_v, acc, sem0, sem1, sem2):
    cid = lax.axis_index("c")
    sid = lax.axis_index("s")
    wid = sid * _NCORES + cid

    @pl.when(sid == 0)
    def _():
        pltpu.sync_copy(z_h, acc)

    pltpu.sync_copy(r_h, r_v)
    plsc.subcore_barrier()

    rv = [r_v[pl.ds(16 * d, 16)] for d in range(4)]
    base0 = wid * _EPW
    sems = [sem0, sem1, sem2]

    def start_gather(c, b):
        base = base0 + c * _CHUNK
        pltpu.sync_copy(gidx_h.at[pl.ds(base, _CHUNK)], idx_g.at[b])
        pltpu.sync_copy(sidx_h.at[pl.ds(base, _CHUNK)], idx_s.at[b])
        pltpu.sync_copy(a_h.at[pl.ds(base, _CHUNK)], a_v.at[b])
        pltpu.make_async_copy(table_h.at[idx_g.at[b]], rows.at[b],
                              sems[b]).start()

    def finish_chunk(b):
        pltpu.make_async_copy(table_h.at[idx_g.at[b]], rows.at[b],
                              sems[b]).wait()
        bi = jnp.full((16,), b, dtype=jnp.int32)

        @plsc.parallel_loop(0, _CHUNK, unroll=8)
        def _(i):
            av = plsc.load_gather(
                a_v, [bi, jnp.full((16,), i, dtype=jnp.int32)])
            for d in range(4):
                seg = rows[b, i, pl.ds(16 * d, 16)]
                rows[b, i, pl.ds(16 * d, 16)] = jnp.maximum(
                    seg + av * rv[d], 0.0)

        pltpu.sync_copy(rows.at[b], acc.at[idx_s.at[b]], add=True)

    start_gather(0, 0)
    start_gather(1, 1)

    def outer(i, carry):
        for q in range(_NBUF):
            c = i * _NBUF + q
            start_gather(c + 2, (q + 2) % _NBUF)
            finish_chunk(q)
        return carry

    lax.fori_loop(0, _MAIN // _NBUF, outer, 0)
    for c in range(_MAIN, _NCHUNK):
        b = c % _NBUF
        if c >= _MAIN + 2:
            start_gather(c, b)
        finish_chunk(b)

    plsc.subcore_barrier()

    @pl.when(sid == 0)
    def _():
        pltpu.sync_copy(acc, out_h.at[cid])


def _edge_pass(table, gidx, sidx, a, r):
    z = jnp.zeros((NC, EMB), jnp.float32)
    f = pl.kernel(
        _edge_pass_body,
        mesh=plsc.VectorSubcoreMesh(core_axis_name="c", subcore_axis_name="s"),
        out_type=jax.ShapeDtypeStruct((_NCORES, NC, EMB), jnp.float32),
        scratch_types=[
            pltpu.VMEM((_NBUF, _CHUNK), jnp.int32),
            pltpu.VMEM((_NBUF, _CHUNK), jnp.int32),
            pltpu.VMEM((_NBUF, _CHUNK), jnp.float32),
            pltpu.VMEM((_NBUF, _CHUNK, EMB), jnp.float32),
            pltpu.VMEM((EMB,), jnp.float32),
            pltpu.VMEM_SHARED((NC, EMB), jnp.float32),
            pltpu.SemaphoreType.DMA,
            pltpu.SemaphoreType.DMA,
            pltpu.SemaphoreType.DMA,
        ],
        compiler_params=pltpu.CompilerParams(needs_layout_passes=False,
                                             use_tc_tiling_on_sc=False),
    )
    return f(table, gidx, sidx, a, r, z)


def _mm(x, w):
    return jax.lax.dot_general(x, w, (((1,), (0,)), ((), ())),
                               preferred_element_type=jnp.float32)


def _mmT(x, w):
    return jax.lax.dot_general(x, w, (((1,), (1,)), ((), ())),
                               preferred_element_type=jnp.float32)


def _mm0(x, w):
    return jax.lax.dot_general(x, w, (((0,), (0,)), ((), ())),
                               preferred_element_type=jnp.float32)


def _prep_body(cf_ref, vf_ref, Wc_ref, bc_ref, Wv_ref, bv_ref, Wm1_ref,
               k1_ref, c0_ref, v0_ref, v0mk_ref):
    c0 = jnp.maximum(_mm(cf_ref[...], Wc_ref[...]) + bc_ref[...], 0.0)
    v0 = jnp.maximum(_mm(vf_ref[...], Wv_ref[...]) + bv_ref[...], 0.0)
    c0_ref[...] = c0
    v0_ref[...] = v0
    v0mk_ref[...] = _mm(v0, Wm1_ref[...]) + k1_ref[...]


def _mid_body(s1p_ref, c0_ref, Wu1_ref, bu1_ref, Wm2_ref, k2_ref, c1mk_ref):
    s1 = s1p_ref[0] + s1p_ref[1]
    c1 = jnp.maximum(c0_ref[...] + _mm(s1, Wu1_ref[...]) + bu1_ref[...], 0.0)
    c1mk_ref[...] = _mm(c1, Wm2_ref[...]) + k2_ref[...]


def _node_body(s2p_ref, v0_ref, b3_ref, Wu2_ref, bu2_ref, Wq_ref, bq_ref,
               tokK_ref, tokV_ref, v1_ref, acc3_ref):
    s2 = s2p_ref[0] + s2p_ref[1]
    v1 = jnp.maximum(v0_ref[...] + _mm(s2, Wu2_ref[...]) + bu2_ref[...], 0.0)
    v1_ref[...] = v1

    q = _mm(v1, Wq_ref[...]) + bq_ref[...]
    s = _mmT(q, tokK_ref[...]) * (1.0 / np.sqrt(TD).astype(np.float32))
    m = jnp.max(s, axis=-1, keepdims=True)
    e = jnp.exp(s - m)
    w = e / jnp.sum(e, axis=-1, keepdims=True)
    ns = _mm(w, tokV_ref[...])

    b = b3_ref[0, 0, :]
    oh = (b[:, None] == lax.broadcasted_iota(jnp.int32, (_TP, NB), 1)
          ).astype(jnp.float32)

    @pl.when(pl.program_id(0) == 0)
    def _():
        acc3_ref[...] = jnp.zeros_like(acc3_ref)

    acc3_ref[0] += _mm0(oh, v1)
    acc3_ref[1] += _mm0(oh, ns)
    acc3_ref[2] += _mm0(oh, jnp.ones((_TP, EMB), jnp.float32))


def _gate_body(acc3_ref, Wg_ref, bg_ref, rw_ref):
    cnt = jnp.maximum(acc3_ref[2], 1.0)
    g_emb = acc3_ref[0] / cnt
    st_emb = acc3_ref[1] / cnt
    gate_in = jnp.concatenate([g_emb, st_emb], axis=-1)
    logits = _mm(gate_in, Wg_ref[...]) + bg_ref[...]

    l = logits
    mask = jnp.zeros_like(l)
    iota = lax.broadcasted_iota(jnp.int32, (NB, NE), 1)
    for _ in range(TOPK):
        m = jnp.max(l, axis=-1, keepdims=True)
        is_max = l == m
        first = jnp.min(jnp.where(is_max, iota, NE), axis=-1, keepdims=True)
        fm = (iota == first).astype(l.dtype)
        mask = mask + fm
        l = jnp.where(fm > 0.5, -1e30, l)

    mx = jnp.max(logits, axis=-1, keepdims=True)
    ex = jnp.exp(logits - mx)
    sm = ex / jnp.sum(ex, axis=-1, keepdims=True)
    rw = sm * mask
    rw_ref[...] = rw / (jnp.sum(rw, axis=-1, keepdims=True) + 1e-12)


def _gelu(x):
    return x * 0.5 * (1.0 + lax.erf(x * np.float32(1.0 / np.sqrt(2.0))))


def _ln_rows(o, g, b):
    mu = jnp.mean(o, axis=-1, keepdims=True)
    var = jnp.mean((o - mu) ** 2, axis=-1, keepdims=True)
    return (o - mu) * jax.lax.rsqrt(var + 1e-5) * g + b


def _moe_body(v1_ref, b3_ref, rw_ref, sW1_ref, sb1_ref, sW2_ref, sb2_ref,
              sg_ref, sbe_ref, dW1_ref, db1_ref, dW2_ref, db2_ref, dg_ref,
              dbe_ref, Wd1_ref, bd1_ref, Wd2_ref, bd2_ref, out_ref, acc_ref):
    x = v1_ref[...]
    b = b3_ref[0, 0, :]
    oh = (b[:, None] == lax.broadcasted_iota(jnp.int32, (_TF, NB), 1)
          ).astype(jnp.float32)
    rw_node = _mm(oh, rw_ref[...])

    sh = jnp.zeros((_TF, EMB), jnp.float32)
    for s in range(2):
        h = _gelu(_mm(x, sW1_ref[s]) + sb1_ref[pl.ds(s, 1), :])
        o = _mm(h, sW2_ref[s]) + sb2_ref[pl.ds(s, 1), :]
        sh = sh + _ln_rows(o, sg_ref[pl.ds(s, 1), :], sbe_ref[pl.ds(s, 1), :])
    acc_ref[...] = x + sh * 0.5

    for e_i in range(NE):
        w_e = rw_node[:, e_i:e_i + 1]

        @pl.when(jnp.max(w_e) > 0.0)
        def _():
            h = _gelu(_mm(x, dW1_ref[e_i]) + db1_ref[pl.ds(e_i, 1), :])
            o = _mm(h, dW2_ref[e_i]) + db2_ref[pl.ds(e_i, 1), :]
            o = _ln_rows(o, dg_ref[pl.ds(e_i, 1), :], dbe_ref[pl.ds(e_i, 1), :])
            acc_ref[...] += w_e * o

    hfin = acc_ref[...]
    hd = jnp.maximum(_mm(hfin, Wd1_ref[...]) + bd1_ref[...], 0.0)
    out_ref[...] = _mm(hd, Wd2_ref[...]) + bd2_ref[...]


def _full(shape):
    return pl.BlockSpec(shape, lambda i: tuple(0 for _ in shape))


def kernel(c_feat, edge_idx, edge_attr, v_feat, batch_idx, params):
    p = params
    ci = edge_idx[0]
    vi = edge_idx[1]
    a = edge_attr[:, 0]

    f32 = jnp.float32
    cf8 = jnp.pad(c_feat, ((0, 0), (0, 4)))
    vf8 = jnp.pad(v_feat, ((0, 0), (0, 2)))
    Wc8 = jnp.pad(p['Wc'], ((0, 4), (0, 0)))
    Wv8 = jnp.pad(p['Wv'], ((0, 2), (0, 0)))

    r1 = p['We'][0] @ p['Wm1']
    k1 = (p['be'] @ p['Wm1'] + p['bm1']).reshape(1, EMB)
    r2 = p['We'][0] @ p['Wm2']
    k2 = (p['be'] @ p['Wm2'] + p['bm2']).reshape(1, EMB)

    nblk = NC // _TP
    grid = (nblk,)

    c0, v0, v0mk = pl.pallas_call(
        _prep_body,
        grid=grid,
        in_specs=[
            pl.BlockSpec((_TP, 8), lambda i: (i, 0)),
            pl.BlockSpec((_TP, 8), lambda i: (i, 0)),
            _full((8, EMB)), _full((1, EMB)),
            _full((8, EMB)), _full((1, EMB)),
            _full((EMB, EMB)), _full((1, EMB)),
        ],
        out_specs=[
            pl.BlockSpec((_TP, EMB), lambda i: (i, 0)),
            pl.BlockSpec((_TP, EMB), lambda i: (i, 0)),
            pl.BlockSpec((_TP, EMB), lambda i: (i, 0)),
        ],
        out_shape=[jax.ShapeDtypeStruct((NC, EMB), f32)] * 3,
    )(cf8, vf8, Wc8, p['bc'].reshape(1, EMB), Wv8, p['bv'].reshape(1, EMB),
      p['Wm1'], k1)

    s1p = _edge_pass(v0mk, vi, ci, a, r1)

    c1mk = pl.pallas_call(
        _mid_body,
        grid=grid,
        in_specs=[
            pl.BlockSpec((_NCORES, _TP, EMB), lambda i: (0, i, 0)),
            pl.BlockSpec((_TP, EMB), lambda i: (i, 0)),
            _full((EMB, EMB)), _full((1, EMB)),
            _full((EMB, EMB)), _full((1, EMB)),
        ],
        out_specs=pl.BlockSpec((_TP, EMB), lambda i: (i, 0)),
        out_shape=jax.ShapeDtypeStruct((NC, EMB), f32),
    )(s1p, c0, p['Wu1'], p['bu1'].reshape(1, EMB), p['Wm2'], k2)

    s2p = _edge_pass(c1mk, ci, vi, a, r2)

    b3p = batch_idx.reshape(nblk, 1, _TP)
    v1, acc3 = pl.pallas_call(
        _node_body,
        grid=grid,
        in_specs=[
            pl.BlockSpec((_NCORES, _TP, EMB), lambda i: (0, i, 0)),
            pl.BlockSpec((_TP, EMB), lambda i: (i, 0)),
            pl.BlockSpec((1, 1, _TP), lambda i: (i, 0, 0)),
            _full((EMB, EMB)), _full((1, EMB)),
            _full((EMB, TD)), _full((1, TD)),
            _full((NT, TD)), _full((NT, TD)),
        ],
        out_specs=[
            pl.BlockSpec((_TP, EMB), lambda i: (i, 0)),
            pl.BlockSpec((3, NB, EMB), lambda i: (0, 0, 0)),
        ],
        out_shape=[
            jax.ShapeDtypeStruct((NV, EMB), f32),
            jax.ShapeDtypeStruct((3, NB, EMB), f32),
        ],
    )(s2p, v0, b3p, p['Wu2'], p['bu2'].reshape(1, EMB), p['Wq'],
      p['bq'].reshape(1, TD), p['tok_K'], p['tok_V'])

    scale = p['alpha'] / TEMP
    WgE = p['Wg'] * scale
    bgE = (p['bg'] * scale + p['ebias']).reshape(1, NE)

    rw = pl.pallas_call(
        _gate_body,
        grid=(1,),
        in_specs=[_full((3, NB, EMB)), _full((EMB + TD, NE)), _full((1, NE))],
        out_specs=_full((NB, NE)),
        out_shape=jax.ShapeDtypeStruct((NB, NE), f32),
    )(acc3, WgE, bgE)

    nblk_f = NV // _TF
    b3f = batch_idx.reshape(nblk_f, 1, _TF)
    Wd2p = jnp.pad(p['Wd2'], ((0, 0), (0, 127)))
    bd2p = jnp.pad(p['bd2'].reshape(1, 1), ((0, 0), (0, 127)))

    outp = pl.pallas_call(
        _moe_body,
        grid=(nblk_f,),
        in_specs=[
            pl.BlockSpec((_TF, EMB), lambda i: (i, 0)),
            pl.BlockSpec((1, 1, _TF), lambda i: (i, 0, 0)),
            _full((NB, NE)),
            _full((KS_, EMB, HID)), _full((KS_, HID)),
            _full((KS_, HID, EMB)), _full((KS_, EMB)),
            _full((KS_, EMB)), _full((KS_, EMB)),
            _full((NE, EMB, HID)), _full((NE, HID)),
            _full((NE, HID, EMB)), _full((NE, EMB)),
            _full((NE, EMB)), _full((NE, EMB)),
            _full((EMB, EMB)), _full((1, EMB)),
            _full((EMB, 128)), _full((1, 128)),
        ],
        out_specs=pl.BlockSpec((_TF, 128), lambda i: (i, 0)),
        out_shape=jax.ShapeDtypeStruct((NV, 128), f32),
        scratch_shapes=[pltpu.VMEM((_TF, EMB), f32)],
    )(v1, b3f, rw, p['sW1'], p['sb1'], p['sW2'], p['sb2'], p['sg'], p['sbe'],
      p['dW1'], p['db1'], p['dW2'], p['db2'], p['dg'], p['dbe'],
      p['Wd1'], p['bd1'].reshape(1, EMB), Wd2p, bd2p)

    return outp[:, 0]
